# Initial kernel scaffold; baseline (speedup 1.0000x reference)
import functools

import jax
import jax.numpy as jnp
from jax import lax
from jax.experimental import pallas as pl
from jax.experimental.pallas import tpu as pltpu
from jax.experimental.pallas import tpu_sc as plsc

N = 10000
D = 128
ALPHA = 0.2
E1 = 256000
E2 = 64000
E = E1 + E2

NC = 2
NS = 16
L = 16

N_PAD = 10240
ROWS_PER_TILE = N_PAD // NS
CHUNK = 128
NUM_CHUNKS = E // CHUNK
CHUNKS_E1 = E1 // CHUNK


def _dot_nt(x, w):
    return lax.dot_general(x, w, (((1,), (1,)), ((), ())),
                           preferred_element_type=jnp.float32)


def _node_scalar_body(x_ref, a_ref, a2_ref, o_ref):
    ra = lax.dot_general(a2_ref[...], a_ref[...], (((1,), (0,)), ((), ())),
                         preferred_element_type=jnp.float32)
    w = jnp.concatenate([ra[:, :D], ra[:, D:2 * D]], axis=0)
    o_ref[...] = _dot_nt(x_ref[...], w)


_node_scalars = pl.pallas_call(
    _node_scalar_body,
    out_shape=jax.ShapeDtypeStruct((N, 2), jnp.float32),
)


def _pw_body(a_ref, a2_ref, ee_ref, o_ref):
    ra = lax.dot_general(a2_ref[...], a_ref[...], (((1,), (0,)), ((), ())),
                         preferred_element_type=jnp.float32)
    o_ref[...] = _dot_nt(ee_ref[...], ra[:, 2 * D:])


_PW_BE = 8000


def _make_pw(rows):
    return pl.pallas_call(
        _pw_body,
        grid=(rows // _PW_BE,),
        in_specs=[
            pl.BlockSpec((D, 3 * D), lambda i: (0, 0)),
            pl.BlockSpec((1, D), lambda i: (0, 0)),
            pl.BlockSpec((_PW_BE, D), lambda i: (i, 0)),
        ],
        out_specs=pl.BlockSpec((_PW_BE, 1), lambda i: (i, 0)),
        out_shape=jax.ShapeDtypeStruct((rows, 1), jnp.float32),
    )


_pw_e1 = _make_pw(E1)
_pw_e2 = _make_pw(E2)


def _sc_edge_body(x_hbm, i0_hbm, i1_hbm, pw_hbm, pu_hbm, pv_hbm, ee1_hbm, ee2_hbm,
                  rs_out, hv_out, hw_out,
                  acc_sh, rs_sh,
                  pu_v, pv_v, i0_v, i1_v, pw_v, g_v, rows_v, rowsum_v,
                  red_v, acc1_v, sem):
    cid = lax.axis_index("c")
    sid = lax.axis_index("s")
    row0 = sid * ROWS_PER_TILE
    zeros16 = jnp.zeros((L,), jnp.float32)

    pltpu.sync_copy(pu_hbm, pu_v)
    pltpu.sync_copy(pv_hbm, pv_v)

    def _z1(j, carry):
        rowsum_v[pl.ds(j * L, L)] = zeros16
        return carry
    lax.fori_loop(0, N_PAD // L, _z1, None)

    def _zr(j, carry):
        for k in range(D // L):
            rows_v[j, pl.ds(k * L, L)] = zeros16
        return carry
    lax.fori_loop(0, CHUNK, _zr, None)
    for q in range(ROWS_PER_TILE // CHUNK):
        pltpu.sync_copy(rows_v, acc_sh.at[pl.ds(row0 + q * CHUNK, CHUNK)])
    plsc.subcore_barrier()

    count = 156 + jnp.where(sid < NUM_CHUNKS - 156 * NS, 1, 0)

    def _chunk(i, carry):
        c = sid + i * NS
        base = c * CHUNK
        pltpu.sync_copy(i0_hbm.at[pl.ds(base, CHUNK)], i0_v)
        pltpu.sync_copy(i1_hbm.at[pl.ds(base, CHUNK)], i1_v)
        pltpu.sync_copy(pw_hbm.at[pl.ds(base, CHUNK)], pw_v)

        @pl.when(cid == 0)
        def _():
            pltpu.async_copy(x_hbm.at[i1_v], rows_v, sem).wait()

        @pl.when(cid == 1)
        def _():
            @pl.when(c < CHUNKS_E1)
            def _():
                pltpu.sync_copy(ee1_hbm.at[pl.ds(base, CHUNK)], rows_v)

            @pl.when(c >= CHUNKS_E1)
            def _():
                pltpu.sync_copy(ee2_hbm.at[pl.ds(base - E1, CHUNK)], rows_v)

        for j8 in range(CHUNK // L):
            sl = pl.ds(j8 * L, L)
            i0_16 = i0_v[sl]
            i1_16 = i1_v[sl]
            s = (plsc.load_gather(pu_v, [i0_16])
                 + plsc.load_gather(pv_v, [i1_16]) + pw_v[sl])
            g16 = jnp.exp(jnp.minimum(-s, -ALPHA * s))
            g_v[sl] = g16

            @pl.when(cid == 0)
            def _(g16=g16, i0_16=i0_16):
                plsc.addupdate_scatter(rowsum_v, [i0_16], g16)

        def _scale(j, carry):
            g = g_v[j]
            for k in range(D // L):
                slk = pl.ds(k * L, L)
                rows_v[j, slk] = rows_v[j, slk] * g
            return carry
        lax.fori_loop(0, CHUNK, _scale, None)

        pltpu.sync_copy(rows_v, acc_sh.at[i0_v], add=True)
        return carry

    lax.fori_loop(0, count, _chunk, None)
    plsc.subcore_barrier()

    rsl = pl.ds(row0, ROWS_PER_TILE)

    @pl.when(cid == 0)
    def _():
        pltpu.sync_copy(acc_sh.at[rsl], hv_out.at[rsl])
        pltpu.sync_copy(rowsum_v, rs_sh.at[sid])

    @pl.when(cid == 1)
    def _():
        pltpu.sync_copy(acc_sh.at[rsl], hw_out.at[rsl])

    plsc.subcore_barrier()

    @pl.when(cid == 0)
    def _():
        pltpu.sync_copy(rs_sh.at[:, rsl], red_v)

        def _red(j, carry):
            slj = pl.ds(j * L, L)
            accv = red_v[0, slj]
            for t in range(1, NS):
                accv = accv + red_v[t, slj]
            acc1_v[slj] = accv
            return carry
        lax.fori_loop(0, ROWS_PER_TILE // L, _red, None)
        pltpu.sync_copy(acc1_v, rs_out.at[rsl])


_sc_edge = functools.partial(
    pl.kernel,
    out_type=(
        jax.ShapeDtypeStruct((N_PAD,), jnp.float32),
        jax.ShapeDtypeStruct((N_PAD, D), jnp.float32),
        jax.ShapeDtypeStruct((N_PAD, D), jnp.float32),
    ),
    mesh=plsc.VectorSubcoreMesh(core_axis_name="c", subcore_axis_name="s",
                                num_cores=NC, num_subcores=NS),
    scratch_types=[
        pltpu.VMEM_SHARED((N_PAD, D), jnp.float32),
        pltpu.VMEM_SHARED((NS, N_PAD), jnp.float32),
        pltpu.VMEM((N_PAD,), jnp.float32),
        pltpu.VMEM((N_PAD,), jnp.float32),
        pltpu.VMEM((CHUNK,), jnp.int32),
        pltpu.VMEM((CHUNK,), jnp.int32),
        pltpu.VMEM((CHUNK,), jnp.float32),
        pltpu.VMEM((CHUNK,), jnp.float32),
        pltpu.VMEM((CHUNK, D), jnp.float32),
        pltpu.VMEM((N_PAD,), jnp.float32),
        pltpu.VMEM((NS, ROWS_PER_TILE), jnp.float32),
        pltpu.VMEM((ROWS_PER_TILE,), jnp.float32),
        pltpu.SemaphoreType.DMA,
    ],
)(_sc_edge_body)


def _final_body(x_ref, a_ref, rs_ref, hv_ref, hw_ref, o_ref):
    a_all = a_ref[...]
    rs = rs_ref[...]
    num = rs * _dot_nt(x_ref[...], a_all[:, :D])
    num = num + _dot_nt(hv_ref[...], a_all[:, D:2 * D])
    num = num + _dot_nt(hw_ref[...], a_all[:, 2 * D:])
    den = jnp.where(rs == 0.0, 1e-12, rs)
    h = num / den
    o_ref[...] = jnp.where(h > 0, h, jnp.expm1(h))


_final = pl.pallas_call(
    _final_body,
    out_shape=jax.ShapeDtypeStruct((N, D), jnp.float32),
)


def kernel(input, edge, edge_embed, edge_list_nhop, edge_embed_nhop, a, a_2):
    x = input
    idx0 = jnp.concatenate([edge[0], edge_list_nhop[0]])
    idx1 = jnp.concatenate([edge[1], edge_list_nhop[1]])
    puv = _node_scalars(x, a, a_2)
    pu = jnp.pad(puv[:, 0], (0, N_PAD - N))
    pv = jnp.pad(puv[:, 1], (0, N_PAD - N))
    pw = jnp.concatenate([_pw_e1(a, a_2, edge_embed)[:, 0],
                          _pw_e2(a, a_2, edge_embed_nhop)[:, 0]])
    rs_pad, hv_pad, hw_pad = _sc_edge(x, idx0, idx1, pw, pu, pv,
                                      edge_embed, edge_embed_nhop)
    return _final(x, a, rs_pad[:N, None], hv_pad[:N], hw_pad[:N])


# speedup vs baseline: 2.9106x; 2.9106x over previous
import functools

import jax
import jax.numpy as jnp
from jax import lax
from jax.experimental import pallas as pl
from jax.experimental.pallas import tpu as pltpu
from jax.experimental.pallas import tpu_sc as plsc

N = 10000
D = 128
ALPHA = 0.2
E1 = 256000
E2 = 64000
E = E1 + E2

NC = 2
NS = 16
L = 16

N_PAD = 10240
ROWS_PER_TILE = N_PAD // NS
RS_ROWS = 80
RS_PT = 8
RS_TILES = RS_ROWS // RS_PT
CHUNK = 64
NUM_CHUNKS = E // CHUNK
CHUNKS_E1 = E1 // CHUNK
BASE_COUNT = NUM_CHUNKS // NS


def _dot_nt(x, w):
    return lax.dot_general(x, w, (((1,), (1,)), ((), ())),
                           preferred_element_type=jnp.float32)


def _node_scalar_body(x_ref, a_ref, a2_ref, o_ref):
    ra = lax.dot_general(a2_ref[...], a_ref[...], (((1,), (0,)), ((), ())),
                         preferred_element_type=jnp.float32)
    w = jnp.concatenate([ra[:, :D], ra[:, D:2 * D]], axis=0)
    o_ref[...] = _dot_nt(x_ref[...], w)


_node_scalars = pl.pallas_call(
    _node_scalar_body,
    out_shape=jax.ShapeDtypeStruct((N, 2), jnp.float32),
)


def _pw_body(a_ref, a2_ref, ee_ref, o_ref):
    ra = lax.dot_general(a2_ref[...], a_ref[...], (((1,), (0,)), ((), ())),
                         preferred_element_type=jnp.float32)
    o_ref[...] = _dot_nt(ee_ref[...], ra[:, 2 * D:])


_PW_BE = 8000


def _make_pw(rows):
    return pl.pallas_call(
        _pw_body,
        grid=(rows // _PW_BE,),
        in_specs=[
            pl.BlockSpec((D, 3 * D), lambda i: (0, 0)),
            pl.BlockSpec((1, D), lambda i: (0, 0)),
            pl.BlockSpec((_PW_BE, D), lambda i: (i, 0)),
        ],
        out_specs=pl.BlockSpec((_PW_BE, 1), lambda i: (i, 0)),
        out_shape=jax.ShapeDtypeStruct((rows, 1), jnp.float32),
    )


_pw_e1 = _make_pw(E1)
_pw_e2 = _make_pw(E2)


def _sc_edge_body(x_hbm, i0_hbm, i1_hbm, pw_hbm, pu_hbm, pv_hbm, ee1_hbm, ee2_hbm,
                  rs_out, hv_out, hw_out,
                  acc_sh, pu_sh, pv_sh,
                  pu_v, pv_v, i0_v, i1_v, pw_v, g_v, rows_v, rowsum_v,
                  red_v, acc1_v, sem):
    cid = lax.axis_index("c")
    sid = lax.axis_index("s")
    row0 = sid * ROWS_PER_TILE
    zeros16 = jnp.zeros((L,), jnp.float32)

    @pl.when(sid == 0)
    def _():
        pltpu.sync_copy(pu_hbm, pu_sh)
        pltpu.sync_copy(pv_hbm, pv_sh)
    plsc.subcore_barrier()
    pltpu.sync_copy(pu_sh, pu_v)
    pltpu.sync_copy(pv_sh, pv_v)

    def _z1(j, carry):
        for k in range(D // L):
            rowsum_v[j, pl.ds(k * L, L)] = zeros16
        return carry
    lax.fori_loop(0, RS_ROWS, _z1, None)

    def _zr(j, carry):
        for k in range(D // L):
            rows_v[j, pl.ds(k * L, L)] = zeros16
        return carry
    lax.fori_loop(0, CHUNK, _zr, None)
    for q in range(ROWS_PER_TILE // CHUNK):
        pltpu.sync_copy(rows_v, acc_sh.at[pl.ds(row0 + q * CHUNK, CHUNK)])
    plsc.subcore_barrier()

    count = BASE_COUNT + jnp.where(sid < NUM_CHUNKS - BASE_COUNT * NS, 1, 0)

    def _chunk(i, carry):
        c = sid + i * NS
        base = c * CHUNK
        pltpu.sync_copy(i0_hbm.at[pl.ds(base, CHUNK)], i0_v)
        pltpu.sync_copy(i1_hbm.at[pl.ds(base, CHUNK)], i1_v)
        pltpu.sync_copy(pw_hbm.at[pl.ds(base, CHUNK)], pw_v)

        @pl.when(cid == 0)
        def _():
            pltpu.async_copy(x_hbm.at[i1_v], rows_v, sem).wait()

        @pl.when(cid == 1)
        def _():
            @pl.when(c < CHUNKS_E1)
            def _():
                pltpu.sync_copy(ee1_hbm.at[pl.ds(base, CHUNK)], rows_v)

            @pl.when(c >= CHUNKS_E1)
            def _():
                pltpu.sync_copy(ee2_hbm.at[pl.ds(base - E1, CHUNK)], rows_v)

        for j8 in range(CHUNK // L):
            sl = pl.ds(j8 * L, L)
            i0_16 = i0_v[sl]
            i1_16 = i1_v[sl]
            s = (plsc.load_gather(pu_v, [i0_16])
                 + plsc.load_gather(pv_v, [i1_16]) + pw_v[sl])
            g16 = jnp.exp(jnp.minimum(-s, -ALPHA * s))
            g_v[sl] = g16

            @pl.when(cid == 0)
            def _(g16=g16, i0_16=i0_16):
                plsc.addupdate_scatter(rowsum_v,
                                       [i0_16 >> 7, i0_16 & (D - 1)], g16)

        def _scale(j8, carry):
            g16 = g_v[pl.ds(j8 * L, L)]
            for e in range(L):
                g = g16[e]
                j = j8 * L + e
                for k in range(D // L):
                    slk = pl.ds(k * L, L)
                    rows_v[j, slk] = rows_v[j, slk] * g
            return carry
        lax.fori_loop(0, CHUNK // L, _scale, None)

        pltpu.sync_copy(rows_v, acc_sh.at[i0_v], add=True)
        return carry

    lax.fori_loop(0, count, _chunk, None)
    plsc.subcore_barrier()

    rsl = pl.ds(row0, ROWS_PER_TILE)

    @pl.when(cid == 0)
    def _():
        pltpu.sync_copy(acc_sh.at[rsl], hv_out.at[rsl])

    @pl.when(cid == 1)
    def _():
        pltpu.sync_copy(acc_sh.at[rsl], hw_out.at[rsl])

    plsc.subcore_barrier()

    @pl.when(cid == 0)
    def _():
        pltpu.sync_copy(rowsum_v, acc_sh.at[pl.ds(sid * RS_ROWS, RS_ROWS)])
        plsc.subcore_barrier()

        @pl.when(sid < RS_TILES)
        def _():
            pltpu.sync_copy(acc_sh.at[pl.ds(sid * RS_PT, RS_PT)], acc1_v)

            def _redt(t, carry):
                pltpu.sync_copy(
                    acc_sh.at[pl.ds(t * RS_ROWS + sid * RS_PT, RS_PT)], red_v)
                for r in range(RS_PT):
                    for k in range(D // L):
                        slk = pl.ds(k * L, L)
                        acc1_v[r, slk] = acc1_v[r, slk] + red_v[r, slk]
                return carry
            lax.fori_loop(1, NS, _redt, None)
            pltpu.sync_copy(acc1_v, rs_out.at[pl.ds(sid * RS_PT, RS_PT)])


_sc_edge = functools.partial(
    pl.kernel,
    out_type=(
        jax.ShapeDtypeStruct((RS_ROWS, D), jnp.float32),
        jax.ShapeDtypeStruct((N_PAD, D), jnp.float32),
        jax.ShapeDtypeStruct((N_PAD, D), jnp.float32),
    ),
    mesh=plsc.VectorSubcoreMesh(core_axis_name="c", subcore_axis_name="s",
                                num_cores=NC, num_subcores=NS),
    compiler_params=pltpu.CompilerParams(needs_layout_passes=False),
    scratch_types=[
        pltpu.VMEM_SHARED((N_PAD, D), jnp.float32),
        pltpu.VMEM_SHARED((N_PAD,), jnp.float32),
        pltpu.VMEM_SHARED((N_PAD,), jnp.float32),
        pltpu.VMEM((N_PAD,), jnp.float32),
        pltpu.VMEM((N_PAD,), jnp.float32),
        pltpu.VMEM((CHUNK,), jnp.int32),
        pltpu.VMEM((CHUNK,), jnp.int32),
        pltpu.VMEM((CHUNK,), jnp.float32),
        pltpu.VMEM((CHUNK,), jnp.float32),
        pltpu.VMEM((CHUNK, D), jnp.float32),
        pltpu.VMEM((RS_ROWS, D), jnp.float32),
        pltpu.VMEM((RS_PT, D), jnp.float32),
        pltpu.VMEM((RS_PT, D), jnp.float32),
        pltpu.SemaphoreType.DMA,
    ],
)(_sc_edge_body)


def _final_body(x_ref, a_ref, rs_ref, hv_ref, hw_ref, o_ref):
    a_all = a_ref[...]
    rs = rs_ref[...]
    num = rs * _dot_nt(x_ref[...], a_all[:, :D])
    num = num + _dot_nt(hv_ref[...], a_all[:, D:2 * D])
    num = num + _dot_nt(hw_ref[...], a_all[:, 2 * D:])
    den = jnp.where(rs == 0.0, 1e-12, rs)
    h = num / den
    o_ref[...] = jnp.where(h > 0, h, jnp.exp(jnp.minimum(h, 0.0)) - 1.0)


_final = pl.pallas_call(
    _final_body,
    out_shape=jax.ShapeDtypeStruct((N, D), jnp.float32),
)


def kernel(input, edge, edge_embed, edge_list_nhop, edge_embed_nhop, a, a_2):
    x = input
    idx0 = jnp.concatenate([edge[0], edge_list_nhop[0]])
    idx1 = jnp.concatenate([edge[1], edge_list_nhop[1]])
    puv = _node_scalars(x, a, a_2)
    pu = jnp.pad(puv[:, 0], (0, N_PAD - N))
    pv = jnp.pad(puv[:, 1], (0, N_PAD - N))
    pw = jnp.concatenate([_pw_e1(a, a_2, edge_embed)[:, 0],
                          _pw_e2(a, a_2, edge_embed_nhop)[:, 0]])
    rs_pad, hv_pad, hw_pad = _sc_edge(x, idx0, idx1, pw, pu, pv,
                                      edge_embed, edge_embed_nhop)
    rs = rs_pad.reshape(-1)[:N, None]
    return _final(x, a, rs, hv_pad[:N], hw_pad[:N])
